# baseline (speedup 1.0000x reference)
import functools

import jax
import jax.numpy as jnp
from jax import lax
from jax.experimental import pallas as pl
from jax.experimental.pallas import tpu as pltpu
from jax.experimental.pallas import tpu_sc as plsc

OBS_DIM = 512
ACTION_DIM = 64
N_AGENTS = 8
HIDDEN_DIM = 1024
ASP_DIM = 256
ASP_HIDDEN = 512

BM = 256
OUT_COLS = 80
NCHUNK = 4


def _fused_body(ba_ref, x_ref, W1_ref, b1_ref, W2_ref, b2_ref,
                Wa1_ref, ba1_ref, Wa2_ref, ba2_ref,
                Wv_ref, bv_ref, Wp1_ref, bp1_ref, Wp2_ref, bp2_ref,
                out_ref):
    f32 = jnp.float32
    bf = jnp.bfloat16
    x = x_ref[...].astype(bf)
    h1 = jnp.maximum(jnp.dot(x, W1_ref[...], preferred_element_type=f32) + b1_ref[...], 0.0)
    h = jnp.maximum(jnp.dot(h1.astype(bf), W2_ref[...], preferred_element_type=f32) + b2_ref[...], 0.0)
    hb = h.astype(bf)
    a1 = jnp.maximum(jnp.dot(hb, Wa1_ref[0], preferred_element_type=f32) + ba1_ref[0], 0.0)
    f = jnp.dot(a1.astype(bf), Wa2_ref[0], preferred_element_type=f32) + ba2_ref[0]
    fb = f.astype(bf)
    p1 = jnp.maximum(
        jnp.dot(hb, Wp1_ref[:HIDDEN_DIM, :], preferred_element_type=f32)
        + jnp.dot(fb, Wp1_ref[HIDDEN_DIM:, :], preferred_element_type=f32)
        + bp1_ref[...], 0.0)
    logits = jnp.dot(p1.astype(bf), Wp2_ref[...], preferred_element_type=f32) + bp2_ref[...]
    value = (jnp.sum(h * Wv_ref[:, :HIDDEN_DIM], axis=1, keepdims=True)
             + jnp.sum(f * Wv_ref[:, HIDDEN_DIM:], axis=1, keepdims=True)
             + bv_ref[0])
    out_ref[...] = jnp.concatenate(
        [logits, jnp.broadcast_to(value, (value.shape[0], OUT_COLS - ACTION_DIM))], axis=1)


def _fused_net(x_sorted, block_agent, W1, b1, W2, b2, Wa1, ba1, Wa2, ba2,
               Wv, bv, Wp1, bp1, Wp2, bp2, *, interpret=False):
    m_pad = x_sorted.shape[0]
    nb = m_pad // BM
    grid_spec = pltpu.PrefetchScalarGridSpec(
        num_scalar_prefetch=1,
        grid=(nb,),
        in_specs=[
            pl.BlockSpec((BM, OBS_DIM), lambda i, ba: (i, 0)),
            pl.BlockSpec((OBS_DIM, HIDDEN_DIM), lambda i, ba: (0, 0)),
            pl.BlockSpec((1, HIDDEN_DIM), lambda i, ba: (0, 0)),
            pl.BlockSpec((HIDDEN_DIM, HIDDEN_DIM), lambda i, ba: (0, 0)),
            pl.BlockSpec((1, HIDDEN_DIM), lambda i, ba: (0, 0)),
            pl.BlockSpec((1, HIDDEN_DIM, ASP_HIDDEN), lambda i, ba: (ba[i], 0, 0)),
            pl.BlockSpec((1, 1, ASP_HIDDEN), lambda i, ba: (ba[i], 0, 0)),
            pl.BlockSpec((1, ASP_HIDDEN, ASP_DIM), lambda i, ba: (ba[i], 0, 0)),
            pl.BlockSpec((1, 1, ASP_DIM), lambda i, ba: (ba[i], 0, 0)),
            pl.BlockSpec((1, HIDDEN_DIM + ASP_DIM), lambda i, ba: (0, 0)),
            pl.BlockSpec(memory_space=pltpu.SMEM),
            pl.BlockSpec((HIDDEN_DIM + ASP_DIM, HIDDEN_DIM), lambda i, ba: (0, 0)),
            pl.BlockSpec((1, HIDDEN_DIM), lambda i, ba: (0, 0)),
            pl.BlockSpec((HIDDEN_DIM, ACTION_DIM), lambda i, ba: (0, 0)),
            pl.BlockSpec((1, ACTION_DIM), lambda i, ba: (0, 0)),
        ],
        out_specs=pl.BlockSpec((BM, OUT_COLS), lambda i, ba: (i, 0)),
    )
    return pl.pallas_call(
        _fused_body,
        grid_spec=grid_spec,
        out_shape=jax.ShapeDtypeStruct((m_pad, OUT_COLS), jnp.float32),
        interpret=interpret,
    )(block_agent, x_sorted,
      W1.astype(jnp.bfloat16), b1.reshape(1, -1),
      W2.astype(jnp.bfloat16), b2.reshape(1, -1),
      Wa1.astype(jnp.bfloat16), ba1.reshape(N_AGENTS, 1, ASP_HIDDEN),
      Wa2.astype(jnp.bfloat16), ba2.reshape(N_AGENTS, 1, ASP_DIM),
      Wv.reshape(1, -1), bv,
      Wp1.astype(jnp.bfloat16), bp1.reshape(1, -1),
      Wp2.astype(jnp.bfloat16), bp2.reshape(1, -1))


def _routing_body(ids_ref, dest_ref, padded_ref):
    f32 = jnp.float32
    ids = ids_ref[...]
    r_lt_c = (lax.broadcasted_iota(jnp.int32, (128, 128), 0)
              < lax.broadcasted_iota(jnp.int32, (128, 128), 1)).astype(f32)
    masks = []
    prefs = []
    row_counts = []
    for a in range(N_AGENTS):
        m_a = (ids == a).astype(f32)
        p_a = jnp.dot(m_a, r_lt_c, preferred_element_type=f32, precision=lax.Precision.DEFAULT)
        masks.append(m_a)
        prefs.append(p_a)
        row_counts.append(p_a[:, 127:128] + m_a[:, 127:128])
    cmat = jnp.concatenate(row_counts, axis=1)
    c_lt_r = (lax.broadcasted_iota(jnp.int32, (128, 128), 1)
              < lax.broadcasted_iota(jnp.int32, (128, 128), 0)).astype(f32)
    cex = jnp.dot(c_lt_r, cmat, preferred_element_type=f32, precision=lax.Precision.DEFAULT)
    tot = cex[127:128, :] + cmat[127:128, :]
    padded = jnp.ceil(tot / BM) * BM
    a_lt_b = (lax.broadcasted_iota(jnp.int32, (N_AGENTS, N_AGENTS), 0)
              < lax.broadcasted_iota(jnp.int32, (N_AGENTS, N_AGENTS), 1)).astype(f32)
    poff = jnp.dot(padded, a_lt_b, preferred_element_type=f32, precision=lax.Precision.DEFAULT)
    base = cex + poff
    dest = jnp.zeros((128, 128), f32)
    for a in range(N_AGENTS):
        dest = dest + masks[a] * (prefs[a] + base[:, a:a + 1])
    dest_ref[...] = dest.astype(jnp.int32)
    padded_ref[...] = padded.astype(jnp.int32)


def _routing(ids, m_pad, *, interpret=False):
    dest2, padded = pl.pallas_call(
        _routing_body,
        grid=(1,),
        in_specs=[pl.BlockSpec((128, 128), lambda i: (0, 0))],
        out_specs=[pl.BlockSpec((128, 128), lambda i: (0, 0)),
                   pl.BlockSpec((1, N_AGENTS), lambda i: (0, 0))],
        out_shape=[jax.ShapeDtypeStruct((128, 128), jnp.int32),
                   jax.ShapeDtypeStruct((1, N_AGENTS), jnp.int32)],
        interpret=interpret,
    )(ids.reshape(128, 128))
    dest = dest2.reshape(-1)
    ends = jnp.cumsum(padded[0])
    nb = m_pad // BM
    block_start = jnp.arange(nb, dtype=jnp.int32) * BM
    block_agent = jnp.minimum(
        jnp.sum((block_start[:, None] >= ends[None, :]).astype(jnp.int32), axis=1),
        N_AGENTS - 1)
    return dest, block_agent


NC = 2
NS = 16
NW = NC * NS
SC_CHUNK = 128


def _sc_scatter_rows(x, dest3, m_pad):
    m, d = x.shape
    n_chunk = m // (NW * SC_CHUNK)
    mesh = plsc.VectorSubcoreMesh(core_axis_name="c", subcore_axis_name="s")

    @functools.partial(
        pl.kernel, mesh=mesh,
        out_type=jax.ShapeDtypeStruct((m_pad, d), jnp.float32),
        scratch_types=[
            pltpu.VMEM((SC_CHUNK,), jnp.int32),
            pltpu.VMEM((SC_CHUNK, d), jnp.float32),
            pltpu.SemaphoreType.DMA,
        ],
    )
    def k(x_hbm, dest_hbm, out_hbm, idx_v, rows_v, sem):
        wid = lax.axis_index("s") * NC + lax.axis_index("c")
        for j in range(n_chunk):
            base = (wid * n_chunk + j) * SC_CHUNK
            pltpu.sync_copy(dest_hbm.at[wid, j], idx_v)
            pltpu.sync_copy(x_hbm.at[pl.ds(base, SC_CHUNK)], rows_v)
            pltpu.async_copy(rows_v, out_hbm.at[idx_v], sem).wait()

    return k(x, dest3)


def kernel(obs, agent_ids, W1, b1, W2, b2, Wa1, ba1, Wa2, ba2, Wv, bv, Wp1, bp1, Wp2, bp2):
    b, n, o = obs.shape
    m = b * n
    m_pad = m + N_AGENTS * BM
    x = obs.reshape(m, o)
    ids = agent_ids.reshape(m).astype(jnp.int32)

    dest, block_agent = _routing(ids, m_pad)

    n_chunk = m // (NW * SC_CHUNK)
    x_sorted = _sc_scatter_rows(x, dest.reshape(NW, n_chunk, SC_CHUNK), m_pad)
    outbuf = _fused_net(x_sorted, block_agent, W1, b1, W2, b2, Wa1, ba1, Wa2, ba2,
                        Wv, bv, Wp1, bp1, Wp2, bp2)
    out = jnp.take(outbuf, dest, axis=0, mode='clip')

    values = out[:, ACTION_DIM].reshape(b, n)
    logits = out[:, :ACTION_DIM].reshape(b, n, ACTION_DIM)
    return (values, logits)


# speedup vs baseline: 1.0190x; 1.0190x over previous
import functools

import jax
import jax.numpy as jnp
from jax import lax
from jax.experimental import pallas as pl
from jax.experimental.pallas import tpu as pltpu
from jax.experimental.pallas import tpu_sc as plsc

OBS_DIM = 512
ACTION_DIM = 64
N_AGENTS = 8
HIDDEN_DIM = 1024
ASP_DIM = 256
ASP_HIDDEN = 512

BM = 512
OUT_COLS = 80
NCHUNK = 4


def _fused_body(ba_ref, x_ref, W1_ref, b1_ref, W2_ref, b2_ref,
                Wa1_ref, ba1_ref, Wa2_ref, ba2_ref,
                Wv_ref, bv_ref, Wp1_ref, bp1_ref, Wp2_ref, bp2_ref,
                out_ref):
    f32 = jnp.float32
    bf = jnp.bfloat16
    x = x_ref[...].astype(bf)
    h1 = jnp.maximum(jnp.dot(x, W1_ref[...], preferred_element_type=f32) + b1_ref[...], 0.0)
    h = jnp.maximum(jnp.dot(h1.astype(bf), W2_ref[...], preferred_element_type=f32) + b2_ref[...], 0.0)
    hb = h.astype(bf)
    a1 = jnp.maximum(jnp.dot(hb, Wa1_ref[0], preferred_element_type=f32) + ba1_ref[0], 0.0)
    f = jnp.dot(a1.astype(bf), Wa2_ref[0], preferred_element_type=f32) + ba2_ref[0]
    fb = f.astype(bf)
    p1 = jnp.maximum(
        jnp.dot(hb, Wp1_ref[:HIDDEN_DIM, :], preferred_element_type=f32)
        + jnp.dot(fb, Wp1_ref[HIDDEN_DIM:, :], preferred_element_type=f32)
        + bp1_ref[...], 0.0)
    logits = jnp.dot(p1.astype(bf), Wp2_ref[...], preferred_element_type=f32) + bp2_ref[...]
    value = (jnp.sum(h * Wv_ref[:, :HIDDEN_DIM], axis=1, keepdims=True)
             + jnp.sum(f * Wv_ref[:, HIDDEN_DIM:], axis=1, keepdims=True)
             + bv_ref[0])
    out_ref[...] = jnp.concatenate(
        [logits, jnp.broadcast_to(value, (value.shape[0], OUT_COLS - ACTION_DIM))], axis=1)


def _fused_net(x_sorted, block_agent, W1, b1, W2, b2, Wa1, ba1, Wa2, ba2,
               Wv, bv, Wp1, bp1, Wp2, bp2, *, interpret=False):
    m_pad = x_sorted.shape[0]
    nb = m_pad // BM
    grid_spec = pltpu.PrefetchScalarGridSpec(
        num_scalar_prefetch=1,
        grid=(nb,),
        in_specs=[
            pl.BlockSpec((BM, OBS_DIM), lambda i, ba: (i, 0)),
            pl.BlockSpec((OBS_DIM, HIDDEN_DIM), lambda i, ba: (0, 0)),
            pl.BlockSpec((1, HIDDEN_DIM), lambda i, ba: (0, 0)),
            pl.BlockSpec((HIDDEN_DIM, HIDDEN_DIM), lambda i, ba: (0, 0)),
            pl.BlockSpec((1, HIDDEN_DIM), lambda i, ba: (0, 0)),
            pl.BlockSpec((1, HIDDEN_DIM, ASP_HIDDEN), lambda i, ba: (ba[i], 0, 0)),
            pl.BlockSpec((1, 1, ASP_HIDDEN), lambda i, ba: (ba[i], 0, 0)),
            pl.BlockSpec((1, ASP_HIDDEN, ASP_DIM), lambda i, ba: (ba[i], 0, 0)),
            pl.BlockSpec((1, 1, ASP_DIM), lambda i, ba: (ba[i], 0, 0)),
            pl.BlockSpec((1, HIDDEN_DIM + ASP_DIM), lambda i, ba: (0, 0)),
            pl.BlockSpec(memory_space=pltpu.SMEM),
            pl.BlockSpec((HIDDEN_DIM + ASP_DIM, HIDDEN_DIM), lambda i, ba: (0, 0)),
            pl.BlockSpec((1, HIDDEN_DIM), lambda i, ba: (0, 0)),
            pl.BlockSpec((HIDDEN_DIM, ACTION_DIM), lambda i, ba: (0, 0)),
            pl.BlockSpec((1, ACTION_DIM), lambda i, ba: (0, 0)),
        ],
        out_specs=pl.BlockSpec((BM, OUT_COLS), lambda i, ba: (i, 0)),
    )
    return pl.pallas_call(
        _fused_body,
        grid_spec=grid_spec,
        out_shape=jax.ShapeDtypeStruct((m_pad, OUT_COLS), jnp.float32),
        interpret=interpret,
    )(block_agent, x_sorted,
      W1.astype(jnp.bfloat16), b1.reshape(1, -1),
      W2.astype(jnp.bfloat16), b2.reshape(1, -1),
      Wa1.astype(jnp.bfloat16), ba1.reshape(N_AGENTS, 1, ASP_HIDDEN),
      Wa2.astype(jnp.bfloat16), ba2.reshape(N_AGENTS, 1, ASP_DIM),
      Wv.reshape(1, -1), bv,
      Wp1.astype(jnp.bfloat16), bp1.reshape(1, -1),
      Wp2.astype(jnp.bfloat16), bp2.reshape(1, -1))


def _routing_body(ids_ref, dest_ref, padded_ref):
    f32 = jnp.float32
    ids = ids_ref[...]
    r_lt_c = (lax.broadcasted_iota(jnp.int32, (128, 128), 0)
              < lax.broadcasted_iota(jnp.int32, (128, 128), 1)).astype(f32)
    masks = []
    prefs = []
    row_counts = []
    for a in range(N_AGENTS):
        m_a = (ids == a).astype(f32)
        p_a = jnp.dot(m_a, r_lt_c, preferred_element_type=f32, precision=lax.Precision.DEFAULT)
        masks.append(m_a)
        prefs.append(p_a)
        row_counts.append(p_a[:, 127:128] + m_a[:, 127:128])
    cmat = jnp.concatenate(row_counts, axis=1)
    c_lt_r = (lax.broadcasted_iota(jnp.int32, (128, 128), 1)
              < lax.broadcasted_iota(jnp.int32, (128, 128), 0)).astype(f32)
    cex = jnp.dot(c_lt_r, cmat, preferred_element_type=f32, precision=lax.Precision.DEFAULT)
    tot = cex[127:128, :] + cmat[127:128, :]
    padded = jnp.ceil(tot / BM) * BM
    a_lt_b = (lax.broadcasted_iota(jnp.int32, (N_AGENTS, N_AGENTS), 0)
              < lax.broadcasted_iota(jnp.int32, (N_AGENTS, N_AGENTS), 1)).astype(f32)
    poff = jnp.dot(padded, a_lt_b, preferred_element_type=f32, precision=lax.Precision.DEFAULT)
    base = cex + poff
    dest = jnp.zeros((128, 128), f32)
    for a in range(N_AGENTS):
        dest = dest + masks[a] * (prefs[a] + base[:, a:a + 1])
    dest_ref[...] = dest.astype(jnp.int32)
    padded_ref[...] = padded.astype(jnp.int32)


def _routing(ids, m_pad, *, interpret=False):
    dest2, padded = pl.pallas_call(
        _routing_body,
        grid=(1,),
        in_specs=[pl.BlockSpec((128, 128), lambda i: (0, 0))],
        out_specs=[pl.BlockSpec((128, 128), lambda i: (0, 0)),
                   pl.BlockSpec((1, N_AGENTS), lambda i: (0, 0))],
        out_shape=[jax.ShapeDtypeStruct((128, 128), jnp.int32),
                   jax.ShapeDtypeStruct((1, N_AGENTS), jnp.int32)],
        interpret=interpret,
    )(ids.reshape(128, 128))
    dest = dest2.reshape(-1)
    ends = jnp.cumsum(padded[0])
    nb = m_pad // BM
    block_start = jnp.arange(nb, dtype=jnp.int32) * BM
    block_agent = jnp.minimum(
        jnp.sum((block_start[:, None] >= ends[None, :]).astype(jnp.int32), axis=1),
        N_AGENTS - 1)
    return dest, block_agent


NC = 2
NS = 16
NW = NC * NS
SC_CHUNK = 128


def _sc_scatter_rows(x, dest3, m_pad):
    m, d = x.shape
    n_chunk = m // (NW * SC_CHUNK)
    mesh = plsc.VectorSubcoreMesh(core_axis_name="c", subcore_axis_name="s")

    @functools.partial(
        pl.kernel, mesh=mesh,
        out_type=jax.ShapeDtypeStruct((m_pad, d), jnp.float32),
        scratch_types=[
            pltpu.VMEM((SC_CHUNK,), jnp.int32),
            pltpu.VMEM((SC_CHUNK, d), jnp.float32),
            pltpu.SemaphoreType.DMA,
        ],
    )
    def k(x_hbm, dest_hbm, out_hbm, idx_v, rows_v, sem):
        wid = lax.axis_index("s") * NC + lax.axis_index("c")
        for j in range(n_chunk):
            base = (wid * n_chunk + j) * SC_CHUNK
            pltpu.sync_copy(dest_hbm.at[wid, j], idx_v)
            pltpu.sync_copy(x_hbm.at[pl.ds(base, SC_CHUNK)], rows_v)
            pltpu.async_copy(rows_v, out_hbm.at[idx_v], sem).wait()

    return k(x, dest3)


def kernel(obs, agent_ids, W1, b1, W2, b2, Wa1, ba1, Wa2, ba2, Wv, bv, Wp1, bp1, Wp2, bp2):
    b, n, o = obs.shape
    m = b * n
    m_pad = m + N_AGENTS * BM
    x = obs.reshape(m, o)
    ids = agent_ids.reshape(m).astype(jnp.int32)

    dest, block_agent = _routing(ids, m_pad)

    n_chunk = m // (NW * SC_CHUNK)
    x_sorted = _sc_scatter_rows(x, dest.reshape(NW, n_chunk, SC_CHUNK), m_pad)
    outbuf = _fused_net(x_sorted, block_agent, W1, b1, W2, b2, Wa1, ba1, Wa2, ba2,
                        Wv, bv, Wp1, bp1, Wp2, bp2)
    out = jnp.take(outbuf, dest, axis=0, mode='clip')

    values = out[:, ACTION_DIM].reshape(b, n)
    logits = out[:, :ACTION_DIM].reshape(b, n, ACTION_DIM)
    return (values, logits)


# speedup vs baseline: 1.0198x; 1.0008x over previous
import functools

import jax
import jax.numpy as jnp
from jax import lax
from jax.experimental import pallas as pl
from jax.experimental.pallas import tpu as pltpu
from jax.experimental.pallas import tpu_sc as plsc

OBS_DIM = 512
ACTION_DIM = 64
N_AGENTS = 8
HIDDEN_DIM = 1024
ASP_DIM = 256
ASP_HIDDEN = 512

BM = 512
OUT_COLS = 80
NCHUNK = 4


def _fused_body(ba_ref, x_ref, W1_ref, b1_ref, W2_ref, b2_ref,
                Wa1_ref, ba1_ref, Wa2_ref, ba2_ref,
                Wv_ref, bv_ref, Wp1_ref, bp1_ref, Wp2_ref, bp2_ref,
                out_l_ref, out_v_ref):
    f32 = jnp.float32
    bf = jnp.bfloat16
    x = x_ref[...].astype(bf)
    h1 = jnp.maximum(jnp.dot(x, W1_ref[...], preferred_element_type=f32) + b1_ref[...], 0.0)
    h = jnp.maximum(jnp.dot(h1.astype(bf), W2_ref[...], preferred_element_type=f32) + b2_ref[...], 0.0)
    hb = h.astype(bf)
    a1 = jnp.maximum(jnp.dot(hb, Wa1_ref[0], preferred_element_type=f32) + ba1_ref[0], 0.0)
    f = jnp.dot(a1.astype(bf), Wa2_ref[0], preferred_element_type=f32) + ba2_ref[0]
    fb = f.astype(bf)
    p1 = jnp.maximum(
        jnp.dot(hb, Wp1_ref[:HIDDEN_DIM, :], preferred_element_type=f32)
        + jnp.dot(fb, Wp1_ref[HIDDEN_DIM:, :], preferred_element_type=f32)
        + bp1_ref[...], 0.0)
    logits = jnp.dot(p1.astype(bf), Wp2_ref[...], preferred_element_type=f32) + bp2_ref[...]
    value = (jnp.sum(h * Wv_ref[:, :HIDDEN_DIM], axis=1, keepdims=True)
             + jnp.sum(f * Wv_ref[:, HIDDEN_DIM:], axis=1, keepdims=True)
             + bv_ref[0])
    out_l_ref[...] = logits
    out_v_ref[...] = jnp.broadcast_to(value, (value.shape[0], 8))


def _fused_net(x_sorted, block_agent, W1, b1, W2, b2, Wa1, ba1, Wa2, ba2,
               Wv, bv, Wp1, bp1, Wp2, bp2, *, interpret=False):
    m_pad = x_sorted.shape[0]
    nb = m_pad // BM
    grid_spec = pltpu.PrefetchScalarGridSpec(
        num_scalar_prefetch=1,
        grid=(nb,),
        in_specs=[
            pl.BlockSpec((BM, OBS_DIM), lambda i, ba: (i, 0)),
            pl.BlockSpec((OBS_DIM, HIDDEN_DIM), lambda i, ba: (0, 0)),
            pl.BlockSpec((1, HIDDEN_DIM), lambda i, ba: (0, 0)),
            pl.BlockSpec((HIDDEN_DIM, HIDDEN_DIM), lambda i, ba: (0, 0)),
            pl.BlockSpec((1, HIDDEN_DIM), lambda i, ba: (0, 0)),
            pl.BlockSpec((1, HIDDEN_DIM, ASP_HIDDEN), lambda i, ba: (ba[i], 0, 0)),
            pl.BlockSpec((1, 1, ASP_HIDDEN), lambda i, ba: (ba[i], 0, 0)),
            pl.BlockSpec((1, ASP_HIDDEN, ASP_DIM), lambda i, ba: (ba[i], 0, 0)),
            pl.BlockSpec((1, 1, ASP_DIM), lambda i, ba: (ba[i], 0, 0)),
            pl.BlockSpec((1, HIDDEN_DIM + ASP_DIM), lambda i, ba: (0, 0)),
            pl.BlockSpec(memory_space=pltpu.SMEM),
            pl.BlockSpec((HIDDEN_DIM + ASP_DIM, HIDDEN_DIM), lambda i, ba: (0, 0)),
            pl.BlockSpec((1, HIDDEN_DIM), lambda i, ba: (0, 0)),
            pl.BlockSpec((HIDDEN_DIM, ACTION_DIM), lambda i, ba: (0, 0)),
            pl.BlockSpec((1, ACTION_DIM), lambda i, ba: (0, 0)),
        ],
        out_specs=[pl.BlockSpec((BM, ACTION_DIM), lambda i, ba: (i, 0)),
                   pl.BlockSpec((BM, 8), lambda i, ba: (i, 0))],
    )
    return pl.pallas_call(
        _fused_body,
        grid_spec=grid_spec,
        out_shape=[jax.ShapeDtypeStruct((m_pad, ACTION_DIM), jnp.float32),
                   jax.ShapeDtypeStruct((m_pad, 8), jnp.float32)],
        interpret=interpret,
    )(block_agent, x_sorted,
      W1.astype(jnp.bfloat16), b1.reshape(1, -1),
      W2.astype(jnp.bfloat16), b2.reshape(1, -1),
      Wa1.astype(jnp.bfloat16), ba1.reshape(N_AGENTS, 1, ASP_HIDDEN),
      Wa2.astype(jnp.bfloat16), ba2.reshape(N_AGENTS, 1, ASP_DIM),
      Wv.reshape(1, -1), bv,
      Wp1.astype(jnp.bfloat16), bp1.reshape(1, -1),
      Wp2.astype(jnp.bfloat16), bp2.reshape(1, -1))


def _routing_body(ids_ref, dest_ref, padded_ref):
    f32 = jnp.float32
    ids = ids_ref[...]
    r_lt_c = (lax.broadcasted_iota(jnp.int32, (128, 128), 0)
              < lax.broadcasted_iota(jnp.int32, (128, 128), 1)).astype(f32)
    masks = []
    prefs = []
    row_counts = []
    for a in range(N_AGENTS):
        m_a = (ids == a).astype(f32)
        p_a = jnp.dot(m_a, r_lt_c, preferred_element_type=f32, precision=lax.Precision.DEFAULT)
        masks.append(m_a)
        prefs.append(p_a)
        row_counts.append(p_a[:, 127:128] + m_a[:, 127:128])
    cmat = jnp.concatenate(row_counts, axis=1)
    c_lt_r = (lax.broadcasted_iota(jnp.int32, (128, 128), 1)
              < lax.broadcasted_iota(jnp.int32, (128, 128), 0)).astype(f32)
    cex = jnp.dot(c_lt_r, cmat, preferred_element_type=f32, precision=lax.Precision.DEFAULT)
    tot = cex[127:128, :] + cmat[127:128, :]
    padded = jnp.ceil(tot / BM) * BM
    a_lt_b = (lax.broadcasted_iota(jnp.int32, (N_AGENTS, N_AGENTS), 0)
              < lax.broadcasted_iota(jnp.int32, (N_AGENTS, N_AGENTS), 1)).astype(f32)
    poff = jnp.dot(padded, a_lt_b, preferred_element_type=f32, precision=lax.Precision.DEFAULT)
    base = cex + poff
    dest = jnp.zeros((128, 128), f32)
    for a in range(N_AGENTS):
        dest = dest + masks[a] * (prefs[a] + base[:, a:a + 1])
    dest_ref[...] = dest.astype(jnp.int32)
    padded_ref[...] = padded.astype(jnp.int32)


def _routing(ids, m_pad, *, interpret=False):
    dest2, padded = pl.pallas_call(
        _routing_body,
        grid=(1,),
        in_specs=[pl.BlockSpec((128, 128), lambda i: (0, 0))],
        out_specs=[pl.BlockSpec((128, 128), lambda i: (0, 0)),
                   pl.BlockSpec((1, N_AGENTS), lambda i: (0, 0))],
        out_shape=[jax.ShapeDtypeStruct((128, 128), jnp.int32),
                   jax.ShapeDtypeStruct((1, N_AGENTS), jnp.int32)],
        interpret=interpret,
    )(ids.reshape(128, 128))
    dest = dest2.reshape(-1)
    ends = jnp.cumsum(padded[0])
    nb = m_pad // BM
    block_start = jnp.arange(nb, dtype=jnp.int32) * BM
    block_agent = jnp.minimum(
        jnp.sum((block_start[:, None] >= ends[None, :]).astype(jnp.int32), axis=1),
        N_AGENTS - 1)
    return dest, block_agent


NC = 2
NS = 16
NW = NC * NS
SC_CHUNK = 128


def _sc_scatter_rows(x, dest3, m_pad):
    m, d = x.shape
    n_chunk = m // (NW * SC_CHUNK)
    mesh = plsc.VectorSubcoreMesh(core_axis_name="c", subcore_axis_name="s")

    @functools.partial(
        pl.kernel, mesh=mesh,
        out_type=jax.ShapeDtypeStruct((m_pad, d), jnp.float32),
        scratch_types=[
            pltpu.VMEM((SC_CHUNK,), jnp.int32),
            pltpu.VMEM((SC_CHUNK, d), jnp.float32),
            pltpu.SemaphoreType.DMA,
        ],
    )
    def k(x_hbm, dest_hbm, out_hbm, idx_v, rows_v, sem):
        wid = lax.axis_index("s") * NC + lax.axis_index("c")
        for j in range(n_chunk):
            base = (wid * n_chunk + j) * SC_CHUNK
            pltpu.sync_copy(dest_hbm.at[wid, j], idx_v)
            pltpu.sync_copy(x_hbm.at[pl.ds(base, SC_CHUNK)], rows_v)
            pltpu.async_copy(rows_v, out_hbm.at[idx_v], sem).wait()

    return k(x, dest3)


def kernel(obs, agent_ids, W1, b1, W2, b2, Wa1, ba1, Wa2, ba2, Wv, bv, Wp1, bp1, Wp2, bp2):
    b, n, o = obs.shape
    m = b * n
    m_pad = m + N_AGENTS * BM
    x = obs.reshape(m, o)
    ids = agent_ids.reshape(m).astype(jnp.int32)

    dest, block_agent = _routing(ids, m_pad)

    n_chunk = m // (NW * SC_CHUNK)
    x_sorted = _sc_scatter_rows(x, dest.reshape(NW, n_chunk, SC_CHUNK), m_pad)
    logits_s, vals_s = _fused_net(x_sorted, block_agent, W1, b1, W2, b2, Wa1, ba1, Wa2, ba2,
                                  Wv, bv, Wp1, bp1, Wp2, bp2)
    logits = jnp.take(logits_s, dest, axis=0, mode='clip').reshape(b, n, ACTION_DIM)
    values = jnp.take(vals_s[:, 0], dest, mode='clip').reshape(b, n)
    return (values, logits)


# speedup vs baseline: 1.1907x; 1.1675x over previous
import functools

import jax
import jax.numpy as jnp
from jax import lax
from jax.experimental import pallas as pl
from jax.experimental.pallas import tpu as pltpu
from jax.experimental.pallas import tpu_sc as plsc

OBS_DIM = 512
ACTION_DIM = 64
N_AGENTS = 8
HIDDEN_DIM = 1024
ASP_DIM = 256
ASP_HIDDEN = 512

BM = 512
OUT_COLS = 80
NCHUNK = 4


def _fused_body(sb_ref, sa_ref, rs_ref, re_ref, fi_ref, la_ref,
                x_ref, W1_ref, b1_ref, W2_ref, b2_ref,
                Wa1_ref, ba1_ref, Wa2_ref, ba2_ref,
                Wv_ref, bv_ref, Wp1_ref, bp1_ref, Wp2_ref, bp2_ref,
                out_l_ref, out_v_ref, h_scr, f_scr):
    f32 = jnp.float32
    bf = jnp.bfloat16
    s = pl.program_id(0)
    is_first = fi_ref[s] == 1
    is_last = la_ref[s] == 1

    @pl.when(is_first)
    def _():
        x = x_ref[...].astype(bf)
        h1 = jnp.maximum(jnp.dot(x, W1_ref[...], preferred_element_type=f32) + b1_ref[...], 0.0)
        h_scr[...] = jnp.maximum(
            jnp.dot(h1.astype(bf), W2_ref[...], preferred_element_type=f32) + b2_ref[...], 0.0)

    h = h_scr[...]
    hb = h.astype(bf)
    a1 = jnp.maximum(jnp.dot(hb, Wa1_ref[0], preferred_element_type=f32) + ba1_ref[0], 0.0)
    fc = jnp.dot(a1.astype(bf), Wa2_ref[0], preferred_element_type=f32) + ba2_ref[0]
    rows = lax.broadcasted_iota(jnp.int32, (BM, 1), 0)
    inband = (rows >= rs_ref[s]) & (rows < re_ref[s])
    fc = jnp.where(inband, fc, 0.0)

    @pl.when(is_first)
    def _():
        f_scr[...] = fc

    @pl.when(jnp.logical_not(is_first))
    def _():
        f_scr[...] = f_scr[...] + fc

    @pl.when(is_last)
    def _():
        f = f_scr[...]
        fb = f.astype(bf)
        p1 = jnp.maximum(
            jnp.dot(hb, Wp1_ref[:HIDDEN_DIM, :], preferred_element_type=f32)
            + jnp.dot(fb, Wp1_ref[HIDDEN_DIM:, :], preferred_element_type=f32)
            + bp1_ref[...], 0.0)
        logits = jnp.dot(p1.astype(bf), Wp2_ref[...], preferred_element_type=f32) + bp2_ref[...]
        value = (jnp.sum(h * Wv_ref[:, :HIDDEN_DIM], axis=1, keepdims=True)
                 + jnp.sum(f * Wv_ref[:, HIDDEN_DIM:], axis=1, keepdims=True)
                 + bv_ref[0])
        out_l_ref[...] = logits
        out_v_ref[...] = jnp.broadcast_to(value, (value.shape[0], 8))


def _fused_net(x_sorted, sched, W1, b1, W2, b2, Wa1, ba1, Wa2, ba2,
               Wv, bv, Wp1, bp1, Wp2, bp2, *, interpret=False):
    m = x_sorted.shape[0]
    n_steps = m // BM + N_AGENTS - 1
    sb, sa, rs, re, fi, la = sched
    grid_spec = pltpu.PrefetchScalarGridSpec(
        num_scalar_prefetch=6,
        grid=(n_steps,),
        in_specs=[
            pl.BlockSpec((BM, OBS_DIM), lambda i, sb, sa, *_: (sb[i], 0)),
            pl.BlockSpec((OBS_DIM, HIDDEN_DIM), lambda i, *_: (0, 0)),
            pl.BlockSpec((1, HIDDEN_DIM), lambda i, *_: (0, 0)),
            pl.BlockSpec((HIDDEN_DIM, HIDDEN_DIM), lambda i, *_: (0, 0)),
            pl.BlockSpec((1, HIDDEN_DIM), lambda i, *_: (0, 0)),
            pl.BlockSpec((1, HIDDEN_DIM, ASP_HIDDEN), lambda i, sb, sa, *_: (sa[i], 0, 0)),
            pl.BlockSpec((1, 1, ASP_HIDDEN), lambda i, sb, sa, *_: (sa[i], 0, 0)),
            pl.BlockSpec((1, ASP_HIDDEN, ASP_DIM), lambda i, sb, sa, *_: (sa[i], 0, 0)),
            pl.BlockSpec((1, 1, ASP_DIM), lambda i, sb, sa, *_: (sa[i], 0, 0)),
            pl.BlockSpec((1, HIDDEN_DIM + ASP_DIM), lambda i, *_: (0, 0)),
            pl.BlockSpec(memory_space=pltpu.SMEM),
            pl.BlockSpec((HIDDEN_DIM + ASP_DIM, HIDDEN_DIM), lambda i, *_: (0, 0)),
            pl.BlockSpec((1, HIDDEN_DIM), lambda i, *_: (0, 0)),
            pl.BlockSpec((HIDDEN_DIM, ACTION_DIM), lambda i, *_: (0, 0)),
            pl.BlockSpec((1, ACTION_DIM), lambda i, *_: (0, 0)),
        ],
        out_specs=[pl.BlockSpec((BM, ACTION_DIM), lambda i, sb, sa, *_: (sb[i], 0)),
                   pl.BlockSpec((BM, 8), lambda i, sb, sa, *_: (sb[i], 0))],
        scratch_shapes=[pltpu.VMEM((BM, HIDDEN_DIM), jnp.float32),
                        pltpu.VMEM((BM, ASP_DIM), jnp.float32)],
    )
    return pl.pallas_call(
        _fused_body,
        grid_spec=grid_spec,
        out_shape=[jax.ShapeDtypeStruct((m, ACTION_DIM), jnp.float32),
                   jax.ShapeDtypeStruct((m, 8), jnp.float32)],
        interpret=interpret,
    )(sb, sa, rs, re, fi, la, x_sorted,
      W1.astype(jnp.bfloat16), b1.reshape(1, -1),
      W2.astype(jnp.bfloat16), b2.reshape(1, -1),
      Wa1.astype(jnp.bfloat16), ba1.reshape(N_AGENTS, 1, ASP_HIDDEN),
      Wa2.astype(jnp.bfloat16), ba2.reshape(N_AGENTS, 1, ASP_DIM),
      Wv.reshape(1, -1), bv,
      Wp1.astype(jnp.bfloat16), bp1.reshape(1, -1),
      Wp2.astype(jnp.bfloat16), bp2.reshape(1, -1))


def _routing_body(ids_ref, dest_ref, padded_ref):
    f32 = jnp.float32
    ids = ids_ref[...]
    r_lt_c = (lax.broadcasted_iota(jnp.int32, (128, 128), 0)
              < lax.broadcasted_iota(jnp.int32, (128, 128), 1)).astype(f32)
    masks = []
    prefs = []
    row_counts = []
    for a in range(N_AGENTS):
        m_a = (ids == a).astype(f32)
        p_a = jnp.dot(m_a, r_lt_c, preferred_element_type=f32, precision=lax.Precision.DEFAULT)
        masks.append(m_a)
        prefs.append(p_a)
        row_counts.append(p_a[:, 127:128] + m_a[:, 127:128])
    cmat = jnp.concatenate(row_counts, axis=1)
    c_lt_r = (lax.broadcasted_iota(jnp.int32, (128, 128), 1)
              < lax.broadcasted_iota(jnp.int32, (128, 128), 0)).astype(f32)
    cex = jnp.dot(c_lt_r, cmat, preferred_element_type=f32, precision=lax.Precision.DEFAULT)
    tot = cex[127:128, :] + cmat[127:128, :]
    a_lt_b = (lax.broadcasted_iota(jnp.int32, (N_AGENTS, N_AGENTS), 0)
              < lax.broadcasted_iota(jnp.int32, (N_AGENTS, N_AGENTS), 1)).astype(f32)
    coff = jnp.dot(tot, a_lt_b, preferred_element_type=f32)
    base = cex + coff
    dest = jnp.zeros((128, 128), f32)
    for a in range(N_AGENTS):
        dest = dest + masks[a] * (prefs[a] + base[:, a:a + 1])
    dest_ref[...] = dest.astype(jnp.int32)
    padded_ref[...] = tot.astype(jnp.int32)


def _routing(ids, *, interpret=False):
    dest2, counts = pl.pallas_call(
        _routing_body,
        grid=(1,),
        in_specs=[pl.BlockSpec((128, 128), lambda i: (0, 0))],
        out_specs=[pl.BlockSpec((128, 128), lambda i: (0, 0)),
                   pl.BlockSpec((1, N_AGENTS), lambda i: (0, 0))],
        out_shape=[jax.ShapeDtypeStruct((128, 128), jnp.int32),
                   jax.ShapeDtypeStruct((1, N_AGENTS), jnp.int32)],
        interpret=interpret,
    )(ids.reshape(128, 128))
    return dest2.reshape(-1), counts[0]


def _schedule(counts, m):
    nb = m // BM
    n_steps = nb + N_AGENTS - 1
    ends = jnp.cumsum(counts)
    starts = ends - counts
    blk0 = jnp.arange(nb, dtype=jnp.int32) * BM
    ov_lo = jnp.maximum(starts[None, :], blk0[:, None])
    ov_hi = jnp.minimum(ends[None, :], (blk0 + BM)[:, None])
    v = (ov_hi > ov_lo)
    vf = v.reshape(-1)
    pos = jnp.where(vf, jnp.cumsum(vf.astype(jnp.int32)) - 1, n_steps)
    bidx = (jnp.arange(nb * N_AGENTS, dtype=jnp.int32) // N_AGENTS)
    aidx = (jnp.arange(nb * N_AGENTS, dtype=jnp.int32) % N_AGENTS)
    csl = jnp.cumsum(v.astype(jnp.int32), axis=1)
    first2 = (v & (csl == 1)).reshape(-1)
    csr = jnp.cumsum(v[:, ::-1].astype(jnp.int32), axis=1)[:, ::-1]
    last2 = (v & (csr == 1)).reshape(-1)
    rs2 = jnp.clip(ov_lo - blk0[:, None], 0, BM).reshape(-1)
    re2 = jnp.clip(ov_hi - blk0[:, None], 0, BM).reshape(-1)

    def put(vals, fill):
        return jnp.full((n_steps,), fill, jnp.int32).at[pos].set(
            vals.astype(jnp.int32), mode='drop')
    return (put(bidx, nb - 1), put(aidx, N_AGENTS - 1), put(rs2, 0),
            put(re2, 0), put(first2, 0), put(last2, 0))


NC = 2
NS = 16
NW = NC * NS
SC_CHUNK = 128


def _sc_scatter_rows(x, dest3, m_pad):
    m, d = x.shape
    n_chunk = m // (NW * SC_CHUNK)
    mesh = plsc.VectorSubcoreMesh(core_axis_name="c", subcore_axis_name="s")

    @functools.partial(
        pl.kernel, mesh=mesh,
        out_type=jax.ShapeDtypeStruct((m_pad, d), jnp.float32),
        scratch_types=[
            pltpu.VMEM((SC_CHUNK,), jnp.int32),
            pltpu.VMEM((SC_CHUNK, d), jnp.float32),
            pltpu.SemaphoreType.DMA,
        ],
    )
    def k(x_hbm, dest_hbm, out_hbm, idx_v, rows_v, sem):
        wid = lax.axis_index("s") * NC + lax.axis_index("c")
        for j in range(n_chunk):
            base = (wid * n_chunk + j) * SC_CHUNK
            pltpu.sync_copy(dest_hbm.at[wid, j], idx_v)
            pltpu.sync_copy(x_hbm.at[pl.ds(base, SC_CHUNK)], rows_v)
            pltpu.async_copy(rows_v, out_hbm.at[idx_v], sem).wait()

    return k(x, dest3)


def kernel(obs, agent_ids, W1, b1, W2, b2, Wa1, ba1, Wa2, ba2, Wv, bv, Wp1, bp1, Wp2, bp2):
    b, n, o = obs.shape
    m = b * n
    x = obs.reshape(m, o)
    ids = agent_ids.reshape(m).astype(jnp.int32)

    dest, counts = _routing(ids)
    sched = _schedule(counts, m)

    n_chunk = m // (NW * SC_CHUNK)
    x_sorted = _sc_scatter_rows(x, dest.reshape(NW, n_chunk, SC_CHUNK), m)
    logits_s, vals_s = _fused_net(x_sorted, sched, W1, b1, W2, b2, Wa1, ba1, Wa2, ba2,
                                  Wv, bv, Wp1, bp1, Wp2, bp2)
    logits = jnp.take(logits_s, dest, axis=0, mode='clip').reshape(b, n, ACTION_DIM)
    values = jnp.take(vals_s[:, 0], dest, mode='clip').reshape(b, n)
    return (values, logits)
